# Initial kernel scaffold; baseline (speedup 1.0000x reference)
import jax
import jax.numpy as jnp
from jax.experimental import pallas as pl


def kernel(xyz, features, W1, b1, W2, b2, W3, b3):
    raise NotImplementedError("write your pallas kernel here")


# speedup vs baseline: 1.0016x; 1.0016x over previous
import functools

import jax
import jax.numpy as jnp
from jax.experimental import pallas as pl
from jax.experimental.pallas import tpu as pltpu

_NPOINT = 1024
_NSAMPLE = 32


def _fps(xyz, npoint):
    B, N, _ = xyz.shape
    bidx = jnp.arange(B)

    def body(i, state):
        idxs, dists, farthest = state
        idxs = idxs.at[:, i].set(farthest)
        centroid = xyz[bidx, farthest]
        d = jnp.sum((xyz - centroid[:, None, :]) ** 2, axis=-1)
        dists = jnp.minimum(dists, d)
        farthest = jnp.argmax(dists, axis=-1).astype(jnp.int32)
        return idxs, dists, farthest

    init = (
        jnp.zeros((B, npoint), dtype=jnp.int32),
        jnp.full((B, N), 1e10, dtype=jnp.float32),
        jnp.zeros((B,), dtype=jnp.int32),
    )
    idxs, _, _ = jax.lax.fori_loop(0, npoint, body, init)
    return idxs


def _mlp_pool_kernel(x_ref, w1_ref, b1_ref, w2_ref, b2_ref, w3_ref, b3_ref,
                     o_ref):
    x = x_ref[...]
    h = jnp.maximum(jnp.dot(x, w1_ref[...], preferred_element_type=jnp.float32)
                    + b1_ref[...], 0.0)
    h = jnp.maximum(jnp.dot(h, w2_ref[...], preferred_element_type=jnp.float32)
                    + b2_ref[...], 0.0)
    h = jnp.maximum(jnp.dot(h, w3_ref[...], preferred_element_type=jnp.float32)
                    + b3_ref[...], 0.0)
    npts = h.shape[0] // _NSAMPLE
    o_ref[...] = jnp.max(h.reshape(npts, _NSAMPLE, h.shape[1]), axis=1)


def _mlp_pool(h, W1, b1, W2, b2, W3, b3):
    B, S, K, C = h.shape
    x = h.reshape(B * S * K, C)
    rows_per_blk = 2048
    pts_per_blk = rows_per_blk // K
    grid = (x.shape[0] // rows_per_blk,)
    out = pl.pallas_call(
        _mlp_pool_kernel,
        grid=grid,
        in_specs=[
            pl.BlockSpec((rows_per_blk, C), lambda i: (i, 0)),
            pl.BlockSpec((C, 32), lambda i: (0, 0)),
            pl.BlockSpec((1, 32), lambda i: (0, 0)),
            pl.BlockSpec((32, 32), lambda i: (0, 0)),
            pl.BlockSpec((1, 32), lambda i: (0, 0)),
            pl.BlockSpec((32, 64), lambda i: (0, 0)),
            pl.BlockSpec((1, 64), lambda i: (0, 0)),
        ],
        out_specs=pl.BlockSpec((pts_per_blk, 64), lambda i: (i, 0)),
        out_shape=jax.ShapeDtypeStruct((B * S, 64), jnp.float32),
    )(x, W1.T, b1[None, :], W2.T, b2[None, :], W3.T, b3[None, :])
    return out.reshape(B, S, 64)


def kernel(xyz, features, W1, b1, W2, b2, W3, b3):
    gather = jax.vmap(lambda p, i: p[i])

    center_idx = _fps(xyz, _NPOINT)
    new_xyz = gather(xyz, center_idx)

    d = (
        jnp.sum(new_xyz ** 2, axis=-1, keepdims=True)
        - 2.0 * jnp.einsum("bsd,bnd->bsn", new_xyz, xyz)
        + jnp.sum(xyz ** 2, axis=-1)[:, None, :]
    )
    _, knn_idx = jax.lax.top_k(-d, _NSAMPLE)

    grouped_xyz = gather(xyz, knn_idx) - new_xyz[:, :, None, :]
    feat_t = jnp.transpose(features, (0, 2, 1))
    grouped_feat = gather(feat_t, knn_idx)
    h = jnp.concatenate([grouped_xyz, grouped_feat], axis=-1)

    new_features = _mlp_pool(h, W1, b1, W2, b2, W3, b3)
    return gather(xyz, center_idx), jnp.transpose(new_features, (0, 2, 1))


# speedup vs baseline: 1.6678x; 1.6651x over previous
import functools

import jax
import jax.numpy as jnp
from jax.experimental import pallas as pl
from jax.experimental.pallas import tpu as pltpu

_NPOINT = 1024
_NSAMPLE = 32


def _fps_kernel(x_ref, y_ref, z_ref, idx_ref, cx_ref, cy_ref, cz_ref):
    X = x_ref[...]
    Y = y_ref[...]
    Z = z_ref[...]
    B = X.shape[0]
    flat = (jax.lax.broadcasted_iota(jnp.int32, (1, 128, 128), 1) * 128
            + jax.lax.broadcasted_iota(jnp.int32, (1, 128, 128), 2))

    def body(i, st):
        dists, far = st
        idx_ref[pl.ds(i, 1), :] = far.reshape(1, B)
        mask = (flat == far).astype(jnp.float32)
        cx = jnp.sum(X * mask, axis=(1, 2), keepdims=True)
        cy = jnp.sum(Y * mask, axis=(1, 2), keepdims=True)
        cz = jnp.sum(Z * mask, axis=(1, 2), keepdims=True)
        cx_ref[pl.ds(i, 1), :] = cx.reshape(1, B)
        cy_ref[pl.ds(i, 1), :] = cy.reshape(1, B)
        cz_ref[pl.ds(i, 1), :] = cz.reshape(1, B)
        dx = X - cx
        dy = Y - cy
        dz = Z - cz
        d = dx * dx + dy * dy
        d = d + dz * dz
        dists = jnp.minimum(dists, d)
        m = jnp.max(dists, axis=(1, 2), keepdims=True)
        far2 = jnp.min(
            jnp.where(dists == m, flat, jnp.int32(1 << 30)),
            axis=(1, 2), keepdims=True).astype(jnp.int32)
        return dists, far2

    init = (jnp.full((B, 128, 128), 1e10, dtype=jnp.float32),
            jnp.zeros((B, 1, 1), dtype=jnp.int32))
    jax.lax.fori_loop(0, _NPOINT, body, init)


def _fps(xyz, npoint):
    B, N, _ = xyz.shape
    planes = [xyz[:, :, c].reshape(B, 128, 128) for c in range(3)]
    idx, cx, cy, cz = pl.pallas_call(
        _fps_kernel,
        out_shape=[
            jax.ShapeDtypeStruct((npoint, B), jnp.int32),
            jax.ShapeDtypeStruct((npoint, B), jnp.float32),
            jax.ShapeDtypeStruct((npoint, B), jnp.float32),
            jax.ShapeDtypeStruct((npoint, B), jnp.float32),
        ],
    )(*planes)
    return idx.T, jnp.stack([cx.T, cy.T, cz.T], axis=-1)


def _mlp_pool_kernel(x_ref, w1_ref, b1_ref, w2_ref, b2_ref, w3_ref, b3_ref,
                     o_ref):
    x = x_ref[...]
    h = jnp.maximum(jnp.dot(x, w1_ref[...], preferred_element_type=jnp.float32)
                    + b1_ref[...], 0.0)
    h = jnp.maximum(jnp.dot(h, w2_ref[...], preferred_element_type=jnp.float32)
                    + b2_ref[...], 0.0)
    h = jnp.maximum(jnp.dot(h, w3_ref[...], preferred_element_type=jnp.float32)
                    + b3_ref[...], 0.0)
    npts = h.shape[0] // _NSAMPLE
    o_ref[...] = jnp.max(h.reshape(npts, _NSAMPLE, h.shape[1]), axis=1)


def _mlp_pool(h, W1, b1, W2, b2, W3, b3):
    B, S, K, C = h.shape
    x = h.reshape(B * S * K, C)
    rows_per_blk = 2048
    pts_per_blk = rows_per_blk // K
    grid = (x.shape[0] // rows_per_blk,)
    out = pl.pallas_call(
        _mlp_pool_kernel,
        grid=grid,
        in_specs=[
            pl.BlockSpec((rows_per_blk, C), lambda i: (i, 0)),
            pl.BlockSpec((C, 32), lambda i: (0, 0)),
            pl.BlockSpec((1, 32), lambda i: (0, 0)),
            pl.BlockSpec((32, 32), lambda i: (0, 0)),
            pl.BlockSpec((1, 32), lambda i: (0, 0)),
            pl.BlockSpec((32, 64), lambda i: (0, 0)),
            pl.BlockSpec((1, 64), lambda i: (0, 0)),
        ],
        out_specs=pl.BlockSpec((pts_per_blk, 64), lambda i: (i, 0)),
        out_shape=jax.ShapeDtypeStruct((B * S, 64), jnp.float32),
    )(x, W1.T, b1[None, :], W2.T, b2[None, :], W3.T, b3[None, :])
    return out.reshape(B, S, 64)


def kernel(xyz, features, W1, b1, W2, b2, W3, b3):
    gather = jax.vmap(lambda p, i: p[i])

    center_idx, new_xyz = _fps(xyz, _NPOINT)

    d = (
        jnp.sum(new_xyz ** 2, axis=-1, keepdims=True)
        - 2.0 * jnp.einsum("bsd,bnd->bsn", new_xyz, xyz)
        + jnp.sum(xyz ** 2, axis=-1)[:, None, :]
    )
    _, knn_idx = jax.lax.top_k(-d, _NSAMPLE)

    grouped_xyz = gather(xyz, knn_idx) - new_xyz[:, :, None, :]
    feat_t = jnp.transpose(features, (0, 2, 1))
    grouped_feat = gather(feat_t, knn_idx)
    h = jnp.concatenate([grouped_xyz, grouped_feat], axis=-1)

    new_features = _mlp_pool(h, W1, b1, W2, b2, W3, b3)
    return new_xyz, jnp.transpose(new_features, (0, 2, 1))


# speedup vs baseline: 3.8299x; 2.2964x over previous
import functools

import jax
import jax.numpy as jnp
from jax.experimental import pallas as pl
from jax.experimental.pallas import tpu as pltpu

_NPOINT = 1024
_NSAMPLE = 32


def _fps_kernel(x_ref, y_ref, z_ref, idx_ref, cx_ref, cy_ref, cz_ref):
    X = x_ref[...]
    Y = y_ref[...]
    Z = z_ref[...]
    B = X.shape[0]
    flat = (jax.lax.broadcasted_iota(jnp.int32, (1, 128, 128), 1) * 128
            + jax.lax.broadcasted_iota(jnp.int32, (1, 128, 128), 2))

    def body(i, st):
        dists, far = st
        idx_ref[pl.ds(i, 1), :] = far.reshape(1, B)
        mask = (flat == far).astype(jnp.float32)
        cx = jnp.sum(X * mask, axis=(1, 2), keepdims=True)
        cy = jnp.sum(Y * mask, axis=(1, 2), keepdims=True)
        cz = jnp.sum(Z * mask, axis=(1, 2), keepdims=True)
        cx_ref[pl.ds(i, 1), :] = cx.reshape(1, B)
        cy_ref[pl.ds(i, 1), :] = cy.reshape(1, B)
        cz_ref[pl.ds(i, 1), :] = cz.reshape(1, B)
        dx = X - cx
        dy = Y - cy
        dz = Z - cz
        d = dx * dx + dy * dy
        d = d + dz * dz
        dists = jnp.minimum(dists, d)
        m = jnp.max(dists, axis=(1, 2), keepdims=True)
        far2 = jnp.min(
            jnp.where(dists == m, flat, jnp.int32(1 << 30)),
            axis=(1, 2), keepdims=True).astype(jnp.int32)
        return dists, far2

    init = (jnp.full((B, 128, 128), 1e10, dtype=jnp.float32),
            jnp.zeros((B, 1, 1), dtype=jnp.int32))
    jax.lax.fori_loop(0, _NPOINT, body, init)


def _fps(xyz, npoint):
    B, N, _ = xyz.shape
    planes = [xyz[:, :, c].reshape(B, 128, 128) for c in range(3)]
    idx, cx, cy, cz = pl.pallas_call(
        _fps_kernel,
        out_shape=[
            jax.ShapeDtypeStruct((npoint, B), jnp.int32),
            jax.ShapeDtypeStruct((npoint, B), jnp.float32),
            jax.ShapeDtypeStruct((npoint, B), jnp.float32),
            jax.ShapeDtypeStruct((npoint, B), jnp.float32),
        ],
    )(*planes)
    return idx.T, jnp.stack([cx.T, cy.T, cz.T], axis=-1)


def _bitonic_stage(v, i, k, j, desc=False):
    M, L = v.shape
    nb = M // (2 * j)
    v4 = v.reshape(nb, 2, j, L)
    i4 = i.reshape(nb, 2, j, L)
    a, b = v4[:, 0], v4[:, 1]
    ia, ib = i4[:, 0], i4[:, 1]
    if k >= M:
        cond = (b <= a) if desc else (a <= b)
    else:
        bit = (jax.lax.broadcasted_iota(jnp.int32, (nb, 1, L), 0)
               * (2 * j)) & k
        cond = (a <= b) == ((bit != 0) if desc else (bit == 0))
    lo = jnp.where(cond, a, b)
    hi = jnp.where(cond, b, a)
    ilo = jnp.where(cond, ia, ib)
    ihi = jnp.where(cond, ib, ia)
    v = jnp.concatenate([lo[:, None], hi[:, None]], axis=1).reshape(M, L)
    i = jnp.concatenate([ilo[:, None], ihi[:, None]], axis=1).reshape(M, L)
    return v, i


def _bitonic_sort32(v, i, desc=False):
    for k in (2, 4, 8, 16, 32):
        j = k // 2
        while j >= 1:
            v, i = _bitonic_stage(v, i, k, j, desc)
            j //= 2
    return v, i


def _merge_sorted(v1, i1, v2d, i2d):
    M = v1.shape[0]
    v = jnp.concatenate([v1, v2d], axis=0)
    i = jnp.concatenate([i1, i2d], axis=0)
    j = M
    while j >= 1:
        v, i = _bitonic_stage(v, i, 2 * M, j)
        j //= 2
    return v[:M], i[:M]


_CHUNK = 32
_QTILE = 128


def _knn_kernel(xyz_ref, qc_ref, sq_ref, idx_out, d_scratch):
    N = xyz_ref.shape[1]
    p = xyz_ref[0]
    sqx = jnp.sum(p * p, axis=1, keepdims=True)
    cross = jnp.dot(p.astype(jnp.bfloat16), qc_ref[0, 0].astype(jnp.bfloat16),
                    preferred_element_type=jnp.float32)
    d_scratch[...] = (sq_ref[0, 0] - 2.0 * cross) + sqx

    def body(c, st):
        rv, ri = st
        dc = d_scratch[pl.ds(c * _CHUNK, _CHUNK), :]
        ic = (jax.lax.broadcasted_iota(jnp.int32, (_CHUNK, _QTILE), 0)
              + c * _CHUNK)
        dc, ic = _bitonic_sort32(dc, ic, desc=True)
        return _merge_sorted(rv, ri, dc, ic)

    rv0 = jnp.full((_CHUNK, _QTILE), jnp.inf, dtype=jnp.float32)
    ri0 = jnp.zeros((_CHUNK, _QTILE), dtype=jnp.int32)
    _, ri = jax.lax.fori_loop(0, N // _CHUNK, body, (rv0, ri0))
    idx_out[0, 0] = ri


def _knn(xyz, new_xyz_t, sq_q):
    B, N, _ = xyz.shape
    S = sq_q.shape[1]
    nt = S // _QTILE
    qc = new_xyz_t.reshape(B, 3, nt, _QTILE).transpose(0, 2, 1, 3)
    sq = sq_q.reshape(B, nt, 1, _QTILE)
    idx = pl.pallas_call(
        _knn_kernel,
        grid=(B, nt),
        in_specs=[
            pl.BlockSpec((1, N, 3), lambda b, t: (b, 0, 0)),
            pl.BlockSpec((1, 1, 3, _QTILE), lambda b, t: (b, t, 0, 0)),
            pl.BlockSpec((1, 1, 1, _QTILE), lambda b, t: (b, t, 0, 0)),
        ],
        out_specs=pl.BlockSpec((1, 1, _CHUNK, _QTILE),
                               lambda b, t: (b, t, 0, 0)),
        out_shape=jax.ShapeDtypeStruct((B, nt, _CHUNK, _QTILE), jnp.int32),
        scratch_shapes=[pltpu.VMEM((N, _QTILE), jnp.float32)],
    )(xyz, qc, sq)
    return idx.transpose(0, 1, 3, 2).reshape(B, S, _NSAMPLE)


def _mlp_pool_kernel(x_ref, w1_ref, b1_ref, w2_ref, b2_ref, w3_ref, b3_ref,
                     o_ref):
    x = x_ref[...]
    h = jnp.maximum(jnp.dot(x, w1_ref[...], preferred_element_type=jnp.float32)
                    + b1_ref[...], 0.0)
    h = jnp.maximum(jnp.dot(h, w2_ref[...], preferred_element_type=jnp.float32)
                    + b2_ref[...], 0.0)
    h = jnp.maximum(jnp.dot(h, w3_ref[...], preferred_element_type=jnp.float32)
                    + b3_ref[...], 0.0)
    npts = h.shape[0] // _NSAMPLE
    o_ref[...] = jnp.max(h.reshape(npts, _NSAMPLE, h.shape[1]), axis=1)


def _mlp_pool(h, W1, b1, W2, b2, W3, b3):
    B, S, K, C = h.shape
    x = h.reshape(B * S * K, C)
    rows_per_blk = 2048
    pts_per_blk = rows_per_blk // K
    grid = (x.shape[0] // rows_per_blk,)
    out = pl.pallas_call(
        _mlp_pool_kernel,
        grid=grid,
        in_specs=[
            pl.BlockSpec((rows_per_blk, C), lambda i: (i, 0)),
            pl.BlockSpec((C, 32), lambda i: (0, 0)),
            pl.BlockSpec((1, 32), lambda i: (0, 0)),
            pl.BlockSpec((32, 32), lambda i: (0, 0)),
            pl.BlockSpec((1, 32), lambda i: (0, 0)),
            pl.BlockSpec((32, 64), lambda i: (0, 0)),
            pl.BlockSpec((1, 64), lambda i: (0, 0)),
        ],
        out_specs=pl.BlockSpec((pts_per_blk, 64), lambda i: (i, 0)),
        out_shape=jax.ShapeDtypeStruct((B * S, 64), jnp.float32),
    )(x, W1.T, b1[None, :], W2.T, b2[None, :], W3.T, b3[None, :])
    return out.reshape(B, S, 64)


def kernel(xyz, features, W1, b1, W2, b2, W3, b3):
    gather = jax.vmap(lambda p, i: p[i])

    center_idx, new_xyz = _fps(xyz, _NPOINT)

    new_xyz_t = jnp.transpose(new_xyz, (0, 2, 1))
    sq_q = jnp.sum(new_xyz ** 2, axis=-1)
    knn_idx = _knn(xyz, new_xyz_t, sq_q)

    grouped_xyz = gather(xyz, knn_idx) - new_xyz[:, :, None, :]
    feat_t = jnp.transpose(features, (0, 2, 1))
    grouped_feat = gather(feat_t, knn_idx)
    h = jnp.concatenate([grouped_xyz, grouped_feat], axis=-1)

    new_features = _mlp_pool(h, W1, b1, W2, b2, W3, b3)
    return new_xyz, jnp.transpose(new_features, (0, 2, 1))
